# baseline (speedup 1.0000x reference)
import jax
import jax.numpy as jnp
from jax.experimental import pallas as pl
from jax.experimental.pallas import tpu as pltpu

NQ = 7
K = 2048
D = 256
TB = 512


def _rvq_block(x_ref, cb_ref, idx_ref, qz_ref):
    r = x_ref[...]
    qz = jnp.zeros_like(r)
    iota = jax.lax.broadcasted_iota(jnp.int32, (TB, K), 1)
    for q in range(NQ):
        cb = cb_ref[q]
        b2 = jnp.sum(cb * cb, axis=1)[None, :]
        a2 = jnp.sum(r * r, axis=1, keepdims=True)
        e = jax.lax.dot_general(r, cb, (((1,), (1,)), ((), ())),
                                preferred_element_type=jnp.float32)
        d2 = (a2 + b2) - 2.0 * e
        dist = jnp.sqrt(jnp.maximum(d2, 0.0))
        m = jnp.min(dist, axis=1, keepdims=True)
        idx = jnp.min(jnp.where(dist == m, iota, K), axis=1)
        onehot = jnp.where(iota == idx[:, None], 1.0, 0.0)
        quant = jax.lax.dot_general(onehot, cb, (((1,), (0,)), ((), ())),
                                    preferred_element_type=jnp.float32,
                                    precision=jax.lax.Precision.HIGHEST)
        idx_ref[0, :, q] = idx
        qz = qz + quant
        r = r - quant
    qz_ref[...] = qz


def kernel(x, codebooks):
    B, T, d = x.shape
    n = B * T
    xf = x.reshape(n, d)
    nblk = n // TB
    grid = (nblk,)
    idx_out, qz = pl.pallas_call(
        _rvq_block,
        grid=grid,
        in_specs=[
            pl.BlockSpec((TB, D), lambda i: (i, 0)),
            pl.BlockSpec((NQ, K, D), lambda i: (0, 0, 0)),
        ],
        out_specs=[
            pl.BlockSpec((1, TB, 8), lambda i: (i, 0, 0)),
            pl.BlockSpec((TB, D), lambda i: (i, 0)),
        ],
        out_shape=[
            jax.ShapeDtypeStruct((nblk, TB, 8), jnp.int32),
            jax.ShapeDtypeStruct((n, D), jnp.float32),
        ],
        compiler_params=pltpu.CompilerParams(
            dimension_semantics=("parallel",),
        ),
    )(xf, codebooks)
    indices = idx_out[:, :, :NQ].transpose(2, 0, 1).reshape(NQ, B, T)
    return indices, qz.reshape(B, T, d)


# speedup vs baseline: 2.3548x; 2.3548x over previous
import jax
import jax.numpy as jnp
from jax.experimental import pallas as pl
from jax.experimental.pallas import tpu as pltpu
from jax.experimental.pallas import tpu_sc as plsc

NQ = 7
K = 2048
D = 256
TB = 512
W = 128


def _dist_argmin(r, cb, idx_ref):
    b2 = jnp.sum(cb * cb, axis=1)[None, :]
    a2 = jnp.sum(r * r, axis=1, keepdims=True)
    e = jax.lax.dot_general(r, cb, (((1,), (1,)), ((), ())),
                            preferred_element_type=jnp.float32)
    d2 = (a2 + b2) - 2.0 * e
    dist = jnp.sqrt(jnp.maximum(d2, 0.0))
    m = jnp.min(dist, axis=1, keepdims=True)
    iota = jax.lax.broadcasted_iota(jnp.int32, (TB, K), 1)
    idx_ref[0, :] = jnp.min(jnp.where(dist == m, iota, K), axis=1)


def _tc_first(x_ref, cb_ref, idx_ref):
    _dist_argmin(x_ref[...], cb_ref[...], idx_ref)


def _tc_mid(r_ref, qprev_ref, cb_ref, idx_ref, rout_ref):
    r = r_ref[...] - qprev_ref[...]
    rout_ref[...] = r
    _dist_argmin(r, cb_ref[...], idx_ref)


def _tc_final(x_ref, r_ref, q_ref, out_ref):
    out_ref[...] = x_ref[...] - (r_ref[...] - q_ref[...])


def _argmin_call(nblk, n, extra_r=False):
    body = _tc_mid if extra_r else _tc_first
    in_specs = [pl.BlockSpec((TB, D), lambda i: (i, 0))]
    if extra_r:
        in_specs.append(pl.BlockSpec((TB, D), lambda i: (i, 0)))
    in_specs.append(pl.BlockSpec((K, D), lambda i: (0, 0)))
    if extra_r:
        out_specs = [pl.BlockSpec((1, TB), lambda i: (0, i)),
                     pl.BlockSpec((TB, D), lambda i: (i, 0))]
        out_shape = [jax.ShapeDtypeStruct((1, n), jnp.int32),
                     jax.ShapeDtypeStruct((n, D), jnp.float32)]
    else:
        out_specs = pl.BlockSpec((1, TB), lambda i: (0, i))
        out_shape = jax.ShapeDtypeStruct((1, n), jnp.int32)
    return pl.pallas_call(
        body,
        grid=(nblk,),
        in_specs=in_specs,
        out_specs=out_specs,
        out_shape=out_shape,
        compiler_params=pltpu.CompilerParams(
            dimension_semantics=("parallel",),
        ),
    )


def _sc_gather(cb, idx, n):
    mesh = plsc.VectorSubcoreMesh(core_axis_name="core",
                                  subcore_axis_name="subcore")

    @pl.kernel(out_type=jax.ShapeDtypeStruct((n, D), jnp.float32), mesh=mesh)
    def gather_kernel(cb_hbm, idx_hbm, quant_hbm):
        def body(idx_vmem, out_vmem):
            pltpu.sync_copy(cb_hbm.at[idx_vmem.at[0]], out_vmem)

        pltpu.emit_pipeline(
            body,
            grid=(n // W,),
            in_specs=[pl.BlockSpec((1, W), lambda i: (0, i))],
            out_specs=[pl.BlockSpec((W, D), lambda i: (i, 0))],
            core_axis_name=("core", "subcore"),
            dimension_semantics=(pltpu.PARALLEL,),
        )(idx_hbm, quant_hbm)

    return gather_kernel(cb, idx)


def kernel(x, codebooks):
    B, T, d = x.shape
    n = B * T
    xf = x.reshape(n, d)
    nblk = n // TB

    first = _argmin_call(nblk, n, extra_r=False)
    mid = _argmin_call(nblk, n, extra_r=True)
    final = pl.pallas_call(
        _tc_final,
        grid=(nblk,),
        in_specs=[pl.BlockSpec((TB, D), lambda i: (i, 0))] * 3,
        out_specs=pl.BlockSpec((TB, D), lambda i: (i, 0)),
        out_shape=jax.ShapeDtypeStruct((n, D), jnp.float32),
        compiler_params=pltpu.CompilerParams(
            dimension_semantics=("parallel",),
        ),
    )

    indices = []
    idx0 = first(xf, codebooks[0])
    indices.append(idx0)
    quant = _sc_gather(codebooks[0], idx0, n)
    r = xf
    for q in range(1, NQ):
        idxq, r = mid(r, quant, codebooks[q])
        indices.append(idxq)
        quant = _sc_gather(codebooks[q], idxq, n)
    quantized = final(xf, r, quant)

    indices = jnp.concatenate(indices, axis=0).reshape(NQ, B, T)
    return indices, quantized.reshape(B, T, d)
